# baseline (speedup 1.0000x reference)
import functools

import jax
import jax.numpy as jnp
from jax import lax
from jax.experimental import pallas as pl
from jax.experimental.pallas import tpu as pltpu
from jax.experimental.pallas import tpu_sc as plsc

N = 1048576
HIDDEN = 64
THRESHOLD = 0.045
TAU = 20.0

R = 128
BLK = R * 128
GRID = N // BLK

NW = 32
CHUNK = N // NW
CH = 16384


def _tc_body(xf_ref, xr_ref, w1_ref, b1_ref, w2_ref, b2_ref, w3_ref, b3_ref,
             outf_ref, idx_ref, dec_ref, nact_ref, carry_ref):
    i = pl.program_id(0)

    @pl.when(i == 0)
    def _init():
        carry_ref[0] = 0.0
        carry_ref[1] = 0.0
        carry_ref[2] = -1.0

    carry_c = carry_ref[0]
    prev_x = carry_ref[1]
    carry_i = carry_ref[2]

    xf = xf_ref[...]
    h1 = jax.nn.relu(xf * w1_ref[...] + b1_ref[...])
    h2 = jax.nn.relu(
        jnp.dot(h1, w2_ref[...], preferred_element_type=jnp.float32)
        + b2_ref[...])
    outf_ref[...] = (
        jnp.dot(h2, w3_ref[...], preferred_element_type=jnp.float32)
        + b3_ref[...])

    xb = xr_ref[...]
    lane = lax.broadcasted_iota(jnp.int32, (R, 128), 1)
    rowi = lax.broadcasted_iota(jnp.int32, (R, 128), 0)

    xp = pltpu.roll(xb, 1, axis=1)
    col0 = pltpu.roll(xb[:, 127:128], 1, axis=0)
    row1 = lax.broadcasted_iota(jnp.int32, (R, 1), 0)
    col0 = jnp.where(row1 == 0, prev_x, col0)
    xp = jnp.where(lane == 0, col0, xp)

    t_f = (jnp.float32(BLK) * i.astype(jnp.float32)
           + (rowi * 128 + lane).astype(jnp.float32))
    m = jnp.abs(xb - xp) > THRESHOLD
    m = jnp.logical_or(m, t_f == 0.0)
    mf = m.astype(jnp.float32)

    ca = lax.broadcasted_iota(jnp.int32, (128, 128), 0)
    cb_ = lax.broadcasted_iota(jnp.int32, (128, 128), 1)
    tri_u = (ca <= cb_).astype(jnp.float32)
    cs = jnp.dot(mf, tri_u, preferred_element_type=jnp.float32)
    rowsum = cs[:, 127:128]
    ra = lax.broadcasted_iota(jnp.int32, (R, R), 0)
    rb = lax.broadcasted_iota(jnp.int32, (R, R), 1)
    tri_l = (rb < ra).astype(jnp.float32)
    ex = jnp.dot(tri_l, rowsum, preferred_element_type=jnp.float32)
    c_blk = cs + ex + carry_c
    carry_c_new = carry_c + jnp.sum(mf)

    dec_ref[...] = jnp.exp((c_blk - 1.0 - t_f) * (1.0 / TAU))

    v = jnp.where(m, t_f, -1.0)
    for s in (1, 2, 4, 8, 16, 32, 64):
        sh = pltpu.roll(v, s, axis=1)
        sh = jnp.where(lane >= s, sh, -1.0)
        v = jnp.maximum(v, sh)
    w = v[:, 127:128]
    for s in (1, 2, 4, 8, 16, 32, 64):
        sh = pltpu.roll(w, s, axis=0)
        sh = jnp.where(row1 >= s, sh, -1.0)
        w = jnp.maximum(w, sh)
    exm = pltpu.roll(w, 1, axis=0)
    exm = jnp.where(row1 >= 1, exm, carry_i)
    ib = jnp.maximum(v, exm)
    idx_ref[...] = ib.astype(jnp.int32)

    carry_ref[0] = carry_c_new
    carry_ref[1] = jnp.sum(xb[R - 1:R, 127:128])
    carry_ref[2] = jnp.max(ib)
    nact_ref[0, 0] = carry_c_new.astype(jnp.int32)


def _tc_stage(x_flat, x_rows, w1r, b1r, w2t, b2r, w3t, b3r):
    return pl.pallas_call(
        _tc_body,
        grid=(GRID,),
        in_specs=[
            pl.BlockSpec((BLK, 1), lambda i: (i, 0)),
            pl.BlockSpec((R, 128), lambda i: (i, 0)),
            pl.BlockSpec((1, HIDDEN), lambda i: (0, 0)),
            pl.BlockSpec((1, HIDDEN), lambda i: (0, 0)),
            pl.BlockSpec((HIDDEN, HIDDEN), lambda i: (0, 0)),
            pl.BlockSpec((1, HIDDEN), lambda i: (0, 0)),
            pl.BlockSpec((HIDDEN, 1), lambda i: (0, 0)),
            pl.BlockSpec((1, 1), lambda i: (0, 0)),
        ],
        out_specs=[
            pl.BlockSpec((BLK, 1), lambda i: (i, 0)),
            pl.BlockSpec((R, 128), lambda i: (i, 0)),
            pl.BlockSpec((R, 128), lambda i: (i, 0)),
            pl.BlockSpec((1, 1), lambda i: (0, 0), memory_space=pltpu.SMEM),
        ],
        out_shape=[
            jax.ShapeDtypeStruct((N, 1), jnp.float32),
            jax.ShapeDtypeStruct((N // 128, 128), jnp.int32),
            jax.ShapeDtypeStruct((N // 128, 128), jnp.float32),
            jax.ShapeDtypeStruct((1, 1), jnp.int32),
        ],
        scratch_shapes=[pltpu.SMEM((3,), jnp.float32)],
    )(x_flat, x_rows, w1r, b1r, w2t, b2r, w3t, b3r)


@functools.cache
def _sc_gather_fn():
    mesh = plsc.VectorSubcoreMesh(
        core_axis_name="c", subcore_axis_name="s", num_cores=2)

    @functools.partial(
        pl.kernel,
        out_type=jax.ShapeDtypeStruct((N,), jnp.float32),
        mesh=mesh,
        scratch_types=[
            pltpu.VMEM((CH,), jnp.int32),
            pltpu.VMEM((CH,), jnp.float32),
            pltpu.VMEM((CH,), jnp.float32),
            pltpu.VMEM((CH,), jnp.float32),
            pltpu.SemaphoreType.DMA,
        ],
    )
    def sc_gather(outf_hbm, idx_hbm, dec_hbm, y_hbm, idx_v, val_v, dec_v,
                  y_v, sem):
        wid = lax.axis_index("s") * 2 + lax.axis_index("c")
        for j in range(CHUNK // CH):
            base = wid * CHUNK + j * CH
            pltpu.sync_copy(idx_hbm.at[pl.ds(base, CH)], idx_v)
            pltpu.async_copy(outf_hbm.at[idx_v], val_v, sem).wait()
            pltpu.sync_copy(dec_hbm.at[pl.ds(base, CH)], dec_v)

            def mul_body(k, _):
                sl = pl.ds(k * 16, 16)
                y_v[sl] = val_v[sl] * dec_v[sl]
                return 0

            lax.fori_loop(0, CH // 16, mul_body, 0)
            pltpu.sync_copy(y_v, y_hbm.at[pl.ds(base, CH)])

    return sc_gather


def kernel(x, W1, b1, W2, b2, W3, b3):
    x_rows = x.reshape(N // 128, 128)
    w1r = W1.reshape(1, HIDDEN)
    b1r = b1.reshape(1, HIDDEN)
    w2t = W2.T
    b2r = b2.reshape(1, HIDDEN)
    w3t = W3.reshape(1, HIDDEN).T
    b3r = b3.reshape(1, 1)

    out_full, idx_rows, dec_rows, nact = _tc_stage(
        x, x_rows, w1r, b1r, w2t, b2r, w3t, b3r)

    y = _sc_gather_fn()(out_full.reshape(N), idx_rows.reshape(N),
                        dec_rows.reshape(N))
    return y.reshape(N, 1), nact[0, 0]
